# baseline (speedup 1.0000x reference)
import functools

import jax
import jax.numpy as jnp
from jax import lax
from jax.experimental import pallas as pl
from jax.experimental.pallas import tpu as pltpu
from jax.experimental.pallas import tpu_sc as plsc

B = 64
C = 1_000_000
MARGIN = 0.2
NEG_INF = float("-inf")

NUM_CORES = 2
NUM_SUBCORES = 16
NW = NUM_CORES * NUM_SUBCORES
ROWS_PER_W = B // NW
CHUNK = 50_000
NCHUNKS = C // CHUNK
NACC = 25
INNER = CHUNK // (16 * NACC)
LANES = 16


def _sc_body(logits_hbm, targets_hbm, out_hbm, t_v, buf0, buf1, out_v,
             sem0, sem1):
    wid = lax.axis_index("s") * NUM_CORES + lax.axis_index("c")
    iota = lax.iota(jnp.int32, LANES)

    pltpu.sync_copy(targets_hbm, t_v)
    tg = plsc.load_gather(t_v, [2 * wid + jnp.minimum(iota, 1)])
    t_rows = [jnp.max(jnp.where(iota == rr, tg, 0)) for rr in range(ROWS_PER_W)]

    bufs = [buf0, buf1]
    sems = [sem0, sem1]
    handles = [None, None]

    def src(k):
        rr, c = divmod(k, NCHUNKS)
        return logits_hbm.at[2 * wid + rr, pl.ds(c * CHUNK, CHUNK)]

    total = ROWS_PER_W * NCHUNKS
    handles[0] = pltpu.async_copy(src(0), bufs[0], sems[0])

    neg_inf_v = jnp.full((LANES,), NEG_INF, jnp.float32)
    losses = []
    accs = None
    t_acc = None
    for k in range(total):
        b = k % 2
        if k + 1 < total:
            handles[(k + 1) % 2] = pltpu.async_copy(
                src(k + 1), bufs[(k + 1) % 2], sems[(k + 1) % 2])
        handles[b].wait()
        buf = bufs[b]
        rr, c = divmod(k, NCHUNKS)
        if c == 0:
            accs = tuple(neg_inf_v for _ in range(NACC))
            t_acc = neg_inf_v
        t = t_rows[rr]

        start = c * CHUNK
        in_range = jnp.logical_and(t >= start, t < start + CHUNK)
        tloc = jnp.clip(t - start, 0, CHUNK - 1)
        idx_vec = jnp.broadcast_to(tloc, (LANES,))
        msk = jnp.logical_and(iota == 0, in_range)
        gathered = plsc.load_gather(buf, [idx_vec], mask=msk)
        t_acc = jnp.maximum(t_acc, jnp.where(msk, gathered, NEG_INF))
        plsc.store_scatter(buf, [idx_vec], neg_inf_v, mask=msk)

        def inner(i, carry):
            base = i * (LANES * NACC)
            out = []
            for u in range(NACC):
                v = buf[pl.ds(base + u * LANES, LANES)]
                out.append(jnp.maximum(carry[u], v))
            return tuple(out)

        accs = lax.fori_loop(0, INNER, inner, accs)

        if c == NCHUNKS - 1:
            m = list(accs)
            while len(m) > 1:
                m = [jnp.maximum(m[2 * i], m[2 * i + 1])
                     for i in range(len(m) // 2)] + m[len(m) // 2 * 2:]
            neg = jnp.max(m[0])
            true_logit = jnp.max(t_acc)
            losses.append(jnp.maximum(MARGIN - true_logit + neg, 0.0))

    lv = jnp.where(iota == 0, losses[0],
                   jnp.where(iota == 1, losses[1], 0.0)).astype(jnp.float32)
    out_v[...] = lv
    pltpu.sync_copy(out_v, out_hbm.at[wid])


@functools.partial(jax.jit, static_argnums=())
def _sc_masked_rowmax(logits, targets):
    mesh = plsc.VectorSubcoreMesh(
        core_axis_name="c", subcore_axis_name="s",
        num_cores=NUM_CORES, num_subcores=NUM_SUBCORES)
    return pl.kernel(
        _sc_body,
        out_type=jax.ShapeDtypeStruct((NW, LANES), jnp.float32),
        mesh=mesh,
        scratch_types=[
            pltpu.VMEM((B,), jnp.int32),
            pltpu.VMEM((CHUNK,), jnp.float32),
            pltpu.VMEM((CHUNK,), jnp.float32),
            pltpu.VMEM((LANES,), jnp.float32),
            pltpu.SemaphoreType.DMA,
            pltpu.SemaphoreType.DMA,
        ],
        compiler_params=pltpu.CompilerParams(
            use_tc_tiling_on_sc=False, needs_layout_passes=False),
    )(logits, targets)


def kernel(logits, targets):
    per_row = _sc_masked_rowmax(logits, targets.astype(jnp.int32))
    return jnp.sum(per_row) * jnp.float32(1.0 / B)


# speedup vs baseline: 35.1462x; 35.1462x over previous
import functools

import jax
import jax.numpy as jnp
from jax import lax
from jax.experimental import pallas as pl
from jax.experimental.pallas import tpu as pltpu
from jax.experimental.pallas import tpu_sc as plsc

B = 64
C = 1_000_000
MARGIN = 0.2
NEG_INF = float("-inf")

NUM_CORES = 2
NUM_SUBCORES = 16
NW = NUM_CORES * NUM_SUBCORES
NGROUPS = 8
NQ = 4
LANE = 128
SUB = 8
NT_FULL = C // LANE
NTQ = NT_FULL // NQ
KT = 31
NCHUNKS = NTQ // KT
TAIL_COL = NT_FULL * LANE
TAIL_W = C - TAIL_COL
L = 16


def _sc_body(logits_hbm, targets_hbm, out_hbm, t_v, buf0, buf1, tail_v, out_v,
             sem0, sem1, sem2):
    wid = lax.axis_index("s") * NUM_CORES + lax.axis_index("c")
    g = wid // NQ
    q = wid % NQ
    row0 = SUB * g
    col_base = q * (NTQ * LANE)
    iota = lax.iota(jnp.int32, L)

    pltpu.sync_copy(targets_hbm, t_v)
    tg = plsc.load_gather(t_v, [row0 + jnp.minimum(iota, SUB - 1)])
    t_r = [jnp.max(jnp.where(iota == r, tg, 0)) for r in range(SUB)]

    tail_cp = pltpu.async_copy(
        logits_hbm.at[pl.ds(row0, SUB), pl.ds(TAIL_COL, TAIL_W)], tail_v, sem2)

    neg_inf_v = jnp.full((L,), NEG_INF, jnp.float32)

    def src(c):
        return logits_hbm.at[pl.ds(row0, SUB),
                             pl.ds(col_base + c * (KT * LANE), KT * LANE)]

    def mask_target(buf, lo, width, t_acc):
        for r in range(SUB):
            in_rng = jnp.logical_and(t_r[r] >= lo, t_r[r] < lo + width)
            lx = jnp.clip(t_r[r] - lo, 0, width - 1)
            ridx = jnp.full((L,), r, jnp.int32)
            cidx = jnp.broadcast_to(lx, (L,))
            msk = jnp.logical_and(iota == r, in_rng)
            got = plsc.load_gather(buf, [ridx, cidx], mask=msk)
            t_acc = jnp.maximum(t_acc, jnp.where(msk, got, NEG_INF))
            plsc.store_scatter(buf, [ridx, cidx], neg_inf_v, mask=msk)
        return t_acc

    def consume(buf, c, carry):
        accs, t_acc = carry
        t_acc = mask_target(buf, col_base + c * (KT * LANE), KT * LANE, t_acc)

        def inner(i, acc):
            out = list(acc)
            for r in range(SUB):
                for v in range(LANE // L):
                    x = buf[r, pl.ds(i * LANE + v * L, L)]
                    out[r] = jnp.maximum(out[r], x)
            return tuple(out)

        return lax.fori_loop(0, KT, inner, accs), t_acc

    pltpu.async_copy(src(0), buf0, sem0)

    def pair_body(j, carry):
        c0 = 2 * j
        pltpu.make_async_copy(src(c0), buf0, sem0).wait()
        pltpu.async_copy(src(c0 + 1), buf1, sem1)
        carry = consume(buf0, c0, carry)
        pltpu.make_async_copy(src(c0 + 1), buf1, sem1).wait()
        pltpu.async_copy(src(c0 + 2), buf0, sem0)
        carry = consume(buf1, c0 + 1, carry)
        return carry

    init = (tuple([neg_inf_v] * SUB), neg_inf_v)
    accs, t_acc = lax.fori_loop(0, (NCHUNKS - 1) // 2, pair_body, init)

    last = NCHUNKS - 1
    pltpu.make_async_copy(src(last), buf0, sem0).wait()
    accs, t_acc = consume(buf0, last, (accs, t_acc))
    accs = list(accs)

    tail_cp.wait()
    t_acc = mask_target(tail_v, TAIL_COL, TAIL_W, t_acc)
    for r in range(SUB):
        for v in range(TAIL_W // L):
            accs[r] = jnp.maximum(accs[r], tail_v[r, pl.ds(v * L, L)])

    out16 = jnp.full((L,), NEG_INF, jnp.float32)
    for r in range(SUB):
        neg_r = jnp.max(accs[r])
        true_r = jnp.max(jnp.where(iota == r, t_acc, NEG_INF))
        out16 = jnp.where(iota == r, neg_r, out16)
        out16 = jnp.where(iota == SUB + r, true_r, out16)
    out_v[...] = out16
    pltpu.sync_copy(out_v, out_hbm.at[pl.ds(wid * L, L)])


@jax.jit
def _sc_partials(logits, targets):
    mesh = plsc.VectorSubcoreMesh(
        core_axis_name="c", subcore_axis_name="s",
        num_cores=NUM_CORES, num_subcores=NUM_SUBCORES)
    return pl.kernel(
        _sc_body,
        out_type=jax.ShapeDtypeStruct((NW * L,), jnp.float32),
        mesh=mesh,
        scratch_types=[
            pltpu.VMEM((B,), jnp.int32),
            pltpu.VMEM((SUB, KT * LANE), jnp.float32),
            pltpu.VMEM((SUB, KT * LANE), jnp.float32),
            pltpu.VMEM((SUB, TAIL_W), jnp.float32),
            pltpu.VMEM((L,), jnp.float32),
            pltpu.SemaphoreType.DMA,
            pltpu.SemaphoreType.DMA,
            pltpu.SemaphoreType.DMA,
        ],
        compiler_params=pltpu.CompilerParams(needs_layout_passes=False),
    )(logits, targets)


def kernel(logits, targets):
    part = _sc_partials(logits, targets.astype(jnp.int32))
    part = part.reshape(NGROUPS, NQ, L)
    neg = jnp.max(part[:, :, :SUB], axis=1).reshape(B)
    true_logit = jnp.max(part[:, :, SUB:], axis=1).reshape(B)
    return jnp.mean(jax.nn.relu(MARGIN - true_logit + neg))


# speedup vs baseline: 49.3382x; 1.4038x over previous
import functools

import jax
import jax.numpy as jnp
from jax import lax
from jax.experimental import pallas as pl
from jax.experimental.pallas import tpu as pltpu
from jax.experimental.pallas import tpu_sc as plsc

B = 64
C = 1_000_000
MARGIN = 0.2
NEG_INF = float("-inf")

NUM_CORES = 2
NUM_SUBCORES = 16
NW = NUM_CORES * NUM_SUBCORES
NGROUPS = 8
NQ = 4
LANE = 128
SUB = 8
NT_FULL = C // LANE
TC_TILES = 3968
SC_COL0 = TC_TILES * LANE
NTQ = (NT_FULL - TC_TILES) // NQ
KT = 31
NCHUNKS = NTQ // KT
TAIL_COL = NT_FULL * LANE
TAIL_W = C - TAIL_COL
L = 16

BC = 16384
TC_COLS = TC_TILES * LANE
TC_GRID = TC_COLS // BC
assert NTQ % KT == 0 and TC_COLS % BC == 0


def _sc_body(logits_hbm, targets_hbm, out_hbm, t_v, buf0, buf1, tail_v, out_v,
             sem0, sem1, sem2):
    wid = lax.axis_index("s") * NUM_CORES + lax.axis_index("c")
    g = wid // NQ
    q = wid % NQ
    row0 = SUB * g
    col_base = SC_COL0 + q * (NTQ * LANE)
    iota = lax.iota(jnp.int32, L)

    pltpu.sync_copy(targets_hbm, t_v)
    tg = plsc.load_gather(t_v, [row0 + jnp.minimum(iota, SUB - 1)])
    t_r = [jnp.max(jnp.where(iota == r, tg, 0)) for r in range(SUB)]

    tail_cp = pltpu.async_copy(
        logits_hbm.at[pl.ds(row0, SUB), pl.ds(TAIL_COL, TAIL_W)], tail_v, sem2)

    neg_inf_v = jnp.full((L,), NEG_INF, jnp.float32)

    def src(c):
        return logits_hbm.at[pl.ds(row0, SUB),
                             pl.ds(col_base + c * (KT * LANE), KT * LANE)]

    def mask_target(buf, lo, width, t_acc):
        for r in range(SUB):
            in_rng = jnp.logical_and(t_r[r] >= lo, t_r[r] < lo + width)
            lx = jnp.clip(t_r[r] - lo, 0, width - 1)
            ridx = jnp.full((L,), r, jnp.int32)
            cidx = jnp.broadcast_to(lx, (L,))
            msk = jnp.logical_and(iota == r, in_rng)
            got = plsc.load_gather(buf, [ridx, cidx], mask=msk)
            t_acc = jnp.maximum(t_acc, jnp.where(msk, got, NEG_INF))
            plsc.store_scatter(buf, [ridx, cidx], neg_inf_v, mask=msk)
        return t_acc

    def consume(buf, c, carry):
        accs, t_acc = carry
        t_acc = mask_target(buf, col_base + c * (KT * LANE), KT * LANE, t_acc)

        def inner(i, acc):
            out = list(acc)
            for r in range(SUB):
                for v in range(LANE // L):
                    x = buf[r, pl.ds(i * LANE + v * L, L)]
                    out[r] = jnp.maximum(out[r], x)
            return tuple(out)

        return lax.fori_loop(0, KT, inner, accs), t_acc

    pltpu.async_copy(src(0), buf0, sem0)

    def pair_body(j, carry):
        c0 = 2 * j
        pltpu.make_async_copy(src(c0), buf0, sem0).wait()
        pltpu.async_copy(src(c0 + 1), buf1, sem1)
        carry = consume(buf0, c0, carry)
        pltpu.make_async_copy(src(c0 + 1), buf1, sem1).wait()
        pltpu.async_copy(src(c0 + 2), buf0, sem0)
        carry = consume(buf1, c0 + 1, carry)
        return carry

    init = (tuple([neg_inf_v] * SUB), neg_inf_v)
    accs, t_acc = lax.fori_loop(0, (NCHUNKS - 1) // 2, pair_body, init)

    last = NCHUNKS - 1
    pltpu.make_async_copy(src(last), buf0, sem0).wait()
    accs, t_acc = consume(buf0, last, (accs, t_acc))
    accs = list(accs)

    tail_cp.wait()
    t_acc = mask_target(tail_v, TAIL_COL, TAIL_W, t_acc)
    for r in range(SUB):
        for v in range(TAIL_W // L):
            accs[r] = jnp.maximum(accs[r], tail_v[r, pl.ds(v * L, L)])

    out16 = jnp.full((L,), NEG_INF, jnp.float32)
    for r in range(SUB):
        neg_r = jnp.max(accs[r])
        true_r = jnp.max(jnp.where(iota == r, t_acc, NEG_INF))
        out16 = jnp.where(iota == r, neg_r, out16)
        out16 = jnp.where(iota == SUB + r, true_r, out16)
    out_v[...] = out16
    pltpu.sync_copy(out_v, out_hbm.at[pl.ds(wid * L, L)])


@jax.jit
def _sc_partials(logits, targets):
    mesh = plsc.VectorSubcoreMesh(
        core_axis_name="c", subcore_axis_name="s",
        num_cores=NUM_CORES, num_subcores=NUM_SUBCORES)
    return pl.kernel(
        _sc_body,
        out_type=jax.ShapeDtypeStruct((NW * L,), jnp.float32),
        mesh=mesh,
        scratch_types=[
            pltpu.VMEM((B,), jnp.int32),
            pltpu.VMEM((SUB, KT * LANE), jnp.float32),
            pltpu.VMEM((SUB, KT * LANE), jnp.float32),
            pltpu.VMEM((SUB, TAIL_W), jnp.float32),
            pltpu.VMEM((L,), jnp.float32),
            pltpu.SemaphoreType.DMA,
            pltpu.SemaphoreType.DMA,
            pltpu.SemaphoreType.DMA,
        ],
        compiler_params=pltpu.CompilerParams(needs_layout_passes=False),
    )(logits, targets)


def _tc_body(t_ref, x_ref, neg_ref, true_ref, acc, tacc, cnt):
    pid = pl.program_id(0)

    @pl.when(pid == 0)
    def _init():
        acc[...] = jnp.full(acc.shape, NEG_INF, jnp.float32)
        tacc[...] = jnp.zeros(tacc.shape, jnp.float32)
        cnt[...] = jnp.zeros(cnt.shape, jnp.float32)

    x = x_ref[...]
    ids = lax.broadcasted_iota(jnp.int32, (B, BC), 1) + pid * BC
    is_t = ids == t_ref[...]
    xm = jnp.where(is_t, NEG_INF, x)
    acc[...] = jnp.maximum(acc[...], jnp.max(xm, axis=1, keepdims=True))
    tacc[...] = tacc[...] + jnp.sum(jnp.where(is_t, x, 0.0), axis=1,
                                    keepdims=True)
    cnt[...] = cnt[...] + jnp.sum(jnp.where(is_t, 1.0, 0.0), axis=1,
                                  keepdims=True)

    @pl.when(pid == pl.num_programs(0) - 1)
    def _fin():
        neg_ref[...] = acc[...]
        true_ref[...] = jnp.where(cnt[...] > 0, tacc[...], NEG_INF)


@jax.jit
def _tc_partials(logits, targets2d):
    return pl.pallas_call(
        _tc_body,
        grid=(TC_GRID,),
        in_specs=[
            pl.BlockSpec((B, 1), lambda i: (0, 0)),
            pl.BlockSpec((B, BC), lambda i: (0, i)),
        ],
        out_specs=[
            pl.BlockSpec((B, 1), lambda i: (0, 0)),
            pl.BlockSpec((B, 1), lambda i: (0, 0)),
        ],
        out_shape=[
            jax.ShapeDtypeStruct((B, 1), jnp.float32),
            jax.ShapeDtypeStruct((B, 1), jnp.float32),
        ],
        scratch_shapes=[
            pltpu.VMEM((B, 1), jnp.float32),
            pltpu.VMEM((B, 1), jnp.float32),
            pltpu.VMEM((B, 1), jnp.float32),
        ],
    )(targets2d, logits)


def kernel(logits, targets):
    targets = targets.astype(jnp.int32)
    part = _sc_partials(logits, targets)
    neg_tc, true_tc = _tc_partials(logits, targets.reshape(B, 1))
    part = part.reshape(NGROUPS, NQ, L)
    neg = jnp.max(part[:, :, :SUB], axis=1).reshape(B)
    true_logit = jnp.max(part[:, :, SUB:], axis=1).reshape(B)
    neg = jnp.maximum(neg, neg_tc.reshape(B))
    true_logit = jnp.maximum(true_logit, true_tc.reshape(B))
    return jnp.mean(jax.nn.relu(MARGIN - true_logit + neg))
